# Initial kernel scaffold; baseline (speedup 1.0000x reference)
import jax
import jax.numpy as jnp
from jax.experimental import pallas as pl


def kernel(g2m_efeat, grid_nfeat, mesh_nfeat, src_idx, dst_idx, W1e, b1e, W2e, b2e, ge, be, W1d, b1d, W2d, b2d, gd, bd, W1s, b1s, W2s, b2s, gs, bs):
    raise NotImplementedError("write your pallas kernel here")


# speedup vs baseline: 1.9906x; 1.9906x over previous
import functools

import jax
import jax.numpy as jnp
from jax import lax
from jax.experimental import pallas as pl
from jax.experimental.pallas import tpu as pltpu
from jax.experimental.pallas import tpu_sc as plsc

NG = 10000
NM = 2500
NE = 160000
D = 256
H = 256

SEG_W = 128
MPAD = 2496 + SEG_W

NC = 2
NS = 16
NW = NC * NS

GBLK = 200
PER_W = NE // NW
NBLK = PER_W // GBLK

BE = 2000
NEB = NE // BE


def _sc_gather(tab_a, tab_b, src_idx, dst_idx):
    @functools.partial(
        pl.kernel,
        mesh=plsc.VectorSubcoreMesh(core_axis_name="c", subcore_axis_name="s"),
        out_type=(
            jax.ShapeDtypeStruct((NE, H), jnp.float32),
            jax.ShapeDtypeStruct((NE, H), jnp.float32),
        ),
        scratch_types=[
            pltpu.VMEM((GBLK,), jnp.int32),
            pltpu.VMEM((GBLK, H), jnp.float32),
            pltpu.SemaphoreType.DMA,
        ],
    )
    def k(a_hbm, b_hbm, si_hbm, di_hbm, oa_hbm, ob_hbm, idx_v, rows_v, sem):
        wid = lax.axis_index("s") * NC + lax.axis_index("c")
        base = wid * PER_W

        @pl.loop(0, NBLK)
        def _(i):
            off = base + i * GBLK
            pltpu.sync_copy(si_hbm.at[pl.ds(off, GBLK)], idx_v)
            pltpu.async_copy(a_hbm.at[idx_v], rows_v, sem).wait()
            pltpu.sync_copy(rows_v, oa_hbm.at[pl.ds(off, GBLK)])
            pltpu.sync_copy(di_hbm.at[pl.ds(off, GBLK)], idx_v)
            pltpu.async_copy(b_hbm.at[idx_v], rows_v, sem).wait()
            pltpu.sync_copy(rows_v, ob_hbm.at[pl.ds(off, GBLK)])

    return k(tab_a, tab_b, src_idx, dst_idx)


def _mm_body(x_ref, w_ref, o_ref):
    o_ref[...] = jnp.dot(x_ref[...], w_ref[...],
                         preferred_element_type=jnp.float32)


def _rowmm(x, w, blk):
    n = x.shape[0]
    return pl.pallas_call(
        _mm_body,
        grid=(n // blk,),
        in_specs=[
            pl.BlockSpec((blk, x.shape[1]), lambda i: (i, 0)),
            pl.BlockSpec((w.shape[0], w.shape[1]), lambda i: (0, 0)),
        ],
        out_specs=pl.BlockSpec((blk, w.shape[1]), lambda i: (i, 0)),
        out_shape=jax.ShapeDtypeStruct((n, w.shape[1]), jnp.float32),
    )(x, w)


def _ln(y, g, b):
    mu = jnp.mean(y, axis=-1, keepdims=True)
    d = y - mu
    var = jnp.mean(d * d, axis=-1, keepdims=True)
    return d * lax.rsqrt(var + 1e-5) * g + b


def _edge_agg_body(x_ref, ag_ref, bg_ref, dst_ref, w1_ref, b1_ref, w2_ref,
                   b2_ref, g_ref, b_ref, agg_ref):
    i = pl.program_id(0)

    @pl.when(i == 0)
    def _():
        agg_ref[...] = jnp.zeros_like(agg_ref)

    h = jnp.dot(x_ref[...], w1_ref[...], preferred_element_type=jnp.float32)
    h = h + ag_ref[...] + bg_ref[...] + b1_ref[...]
    h = h * jax.nn.sigmoid(h)
    y = jnp.dot(h, w2_ref[...], preferred_element_type=jnp.float32) + b2_ref[...]
    ef = _ln(y, g_ref[...], b_ref[...])

    dstv = dst_ref[0]
    d_lo = jnp.min(dstv)
    d_hi = jnp.max(dstv)
    start = (d_lo // 8) * 8
    nchunk = (d_hi - start) // SEG_W + 1

    def chunk(c, _):
        row0 = start + c * SEG_W
        rows = lax.broadcasted_iota(jnp.int32, (SEG_W, BE), 0) + row0
        oh = (rows == dstv).astype(jnp.float32)
        part = jnp.dot(oh, ef, preferred_element_type=jnp.float32)
        agg_ref[pl.ds(row0, SEG_W), :] += part
        return 0

    lax.fori_loop(0, nchunk, chunk, 0)


def _edge_agg(g2m, ag, bg, dst3, w1a, b1, w2, b2, g, b):
    vspec = lambda r, c: pl.BlockSpec((r, c), lambda i: (0, 0))
    return pl.pallas_call(
        _edge_agg_body,
        grid=(NEB,),
        in_specs=[
            pl.BlockSpec((BE, D), lambda i: (i, 0)),
            pl.BlockSpec((BE, H), lambda i: (i, 0)),
            pl.BlockSpec((BE, H), lambda i: (i, 0)),
            pl.BlockSpec((1, 1, BE), lambda i: (i, 0, 0)),
            vspec(D, H), vspec(1, H), vspec(H, D), vspec(1, D),
            vspec(1, D), vspec(1, D),
        ],
        out_specs=pl.BlockSpec((MPAD, D), lambda i: (0, 0)),
        out_shape=jax.ShapeDtypeStruct((MPAD, D), jnp.float32),
    )(g2m, ag, bg, dst3, w1a, b1, w2, b2, g, b)


def _node_body(x_ref, w1_ref, b1_ref, w2_ref, b2_ref, g_ref, b_ref, o_ref):
    x = x_ref[...]
    h = jnp.dot(x, w1_ref[...], preferred_element_type=jnp.float32) + b1_ref[...]
    h = h * jax.nn.sigmoid(h)
    y = jnp.dot(h, w2_ref[...], preferred_element_type=jnp.float32) + b2_ref[...]
    o_ref[...] = x + _ln(y, g_ref[...], b_ref[...])


def _node_mlp(x, w1, b1, w2, b2, g, b, blk):
    n = x.shape[0]
    vspec = lambda r, c: pl.BlockSpec((r, c), lambda i: (0, 0))
    return pl.pallas_call(
        _node_body,
        grid=(n // blk,),
        in_specs=[
            pl.BlockSpec((blk, D), lambda i: (i, 0)),
            vspec(D, H), vspec(1, H), vspec(H, D), vspec(1, D),
            vspec(1, D), vspec(1, D),
        ],
        out_specs=pl.BlockSpec((blk, D), lambda i: (i, 0)),
        out_shape=jax.ShapeDtypeStruct((n, D), jnp.float32),
    )(x, w1, b1, w2, b2, g, b)


def _mesh_body(x_ref, agg_ref, w1a_ref, w1b_ref, b1_ref, w2_ref, b2_ref,
               g_ref, b_ref, o_ref):
    x = x_ref[...]
    h = (jnp.dot(x, w1a_ref[...], preferred_element_type=jnp.float32)
         + jnp.dot(agg_ref[...], w1b_ref[...], preferred_element_type=jnp.float32)
         + b1_ref[...])
    h = h * jax.nn.sigmoid(h)
    y = jnp.dot(h, w2_ref[...], preferred_element_type=jnp.float32) + b2_ref[...]
    o_ref[...] = x + _ln(y, g_ref[...], b_ref[...])


def _mesh_mlp(xpad, agg, w1a, w1b, b1, w2, b2, g, b):
    vspec = lambda r, c: pl.BlockSpec((r, c), lambda i: (0, 0))
    return pl.pallas_call(
        _mesh_body,
        grid=(1,),
        in_specs=[
            vspec(MPAD, D), vspec(MPAD, D),
            vspec(D, H), vspec(D, H), vspec(1, H), vspec(H, D), vspec(1, D),
            vspec(1, D), vspec(1, D),
        ],
        out_specs=pl.BlockSpec((MPAD, D), lambda i: (0, 0)),
        out_shape=jax.ShapeDtypeStruct((MPAD, D), jnp.float32),
    )(xpad, agg, w1a, w1b, b1, w2, b2, g, b)


def kernel(g2m_efeat, grid_nfeat, mesh_nfeat, src_idx, dst_idx,
           W1e, b1e, W2e, b2e, ge, be,
           W1d, b1d, W2d, b2d, gd, bd,
           W1s, b1s, W2s, b2s, gs, bs):
    r1 = lambda v: v.reshape(1, -1)

    tab_a = _rowmm(grid_nfeat, W1e[D:2 * D], 2000)
    tab_b = _rowmm(mesh_nfeat, W1e[2 * D:], 2500)

    ag, bg = _sc_gather(tab_a, tab_b, src_idx, dst_idx)

    dst3 = dst_idx.reshape(NEB, 1, BE)
    agg = _edge_agg(g2m_efeat, ag, bg, dst3, W1e[:D], r1(b1e), W2e, r1(b2e),
                    r1(ge), r1(be))

    xpad = jnp.pad(mesh_nfeat, ((0, MPAD - NM), (0, 0)))
    mesh_out = _mesh_mlp(xpad, agg, W1d[:D], W1d[D:], r1(b1d),
                         W2d, r1(b2d), r1(gd), r1(bd))[:NM]
    grid_out = _node_mlp(grid_nfeat, W1s, r1(b1s), W2s, r1(b2s),
                         r1(gs), r1(bs), 2000)
    return (grid_out, mesh_out)


# speedup vs baseline: 1.9945x; 1.0020x over previous
import functools

import jax
import jax.numpy as jnp
from jax import lax
from jax.experimental import pallas as pl
from jax.experimental.pallas import tpu as pltpu
from jax.experimental.pallas import tpu_sc as plsc

NG = 10000
NM = 2500
NE = 160000
D = 256
H = 256

SEG_W = 128
MPAD = 2496 + SEG_W

NC = 2
NS = 16
NW = NC * NS

GBLK = 200
PER_W = NE // NW
NBLK = PER_W // GBLK

BE = 2000
NEB = NE // BE


def _sc_gather(tab_a, tab_b, src_idx, dst_idx):
    @functools.partial(
        pl.kernel,
        mesh=plsc.VectorSubcoreMesh(core_axis_name="c", subcore_axis_name="s"),
        out_type=(
            jax.ShapeDtypeStruct((NE, H), jnp.float32),
            jax.ShapeDtypeStruct((NE, H), jnp.float32),
        ),
        scratch_types=[
            pltpu.VMEM((GBLK,), jnp.int32),
            pltpu.VMEM((GBLK, H), jnp.float32),
            pltpu.SemaphoreType.DMA,
        ],
    )
    def k(a_hbm, b_hbm, si_hbm, di_hbm, oa_hbm, ob_hbm, idx_v, rows_v, sem):
        wid = lax.axis_index("s") * NC + lax.axis_index("c")
        base = wid * PER_W

        @pl.loop(0, NBLK)
        def _(i):
            off = base + i * GBLK
            pltpu.sync_copy(si_hbm.at[pl.ds(off, GBLK)], idx_v)
            pltpu.async_copy(a_hbm.at[idx_v], rows_v, sem).wait()
            pltpu.sync_copy(rows_v, oa_hbm.at[pl.ds(off, GBLK)])
            pltpu.sync_copy(di_hbm.at[pl.ds(off, GBLK)], idx_v)
            pltpu.async_copy(b_hbm.at[idx_v], rows_v, sem).wait()
            pltpu.sync_copy(rows_v, ob_hbm.at[pl.ds(off, GBLK)])

    return k(tab_a, tab_b, src_idx, dst_idx)


def _mm_body(x_ref, w_ref, o_ref):
    o_ref[...] = jnp.dot(x_ref[...].astype(jnp.bfloat16), w_ref[...],
                         preferred_element_type=jnp.float32)


def _rowmm(x, w, blk):
    n = x.shape[0]
    return pl.pallas_call(
        _mm_body,
        grid=(n // blk,),
        in_specs=[
            pl.BlockSpec((blk, x.shape[1]), lambda i: (i, 0)),
            pl.BlockSpec((w.shape[0], w.shape[1]), lambda i: (0, 0)),
        ],
        out_specs=pl.BlockSpec((blk, w.shape[1]), lambda i: (i, 0)),
        out_shape=jax.ShapeDtypeStruct((n, w.shape[1]), jnp.float32),
    )(x, w)


def _ln(y, g, b):
    mu = jnp.mean(y, axis=-1, keepdims=True)
    d = y - mu
    var = jnp.mean(d * d, axis=-1, keepdims=True)
    return d * lax.rsqrt(var + 1e-5) * g + b


def _edge_agg_body(x_ref, ag_ref, bg_ref, dst_ref, w1_ref, b1_ref, w2_ref,
                   b2_ref, g_ref, b_ref, agg_ref):
    i = pl.program_id(0)

    @pl.when(i == 0)
    def _():
        agg_ref[...] = jnp.zeros_like(agg_ref)

    h = jnp.dot(x_ref[...].astype(jnp.bfloat16), w1_ref[...],
                preferred_element_type=jnp.float32)
    h = h + ag_ref[...] + bg_ref[...] + b1_ref[...]
    h = h * jax.nn.sigmoid(h)
    y = jnp.dot(h.astype(jnp.bfloat16), w2_ref[...],
                preferred_element_type=jnp.float32) + b2_ref[...]
    ef = _ln(y, g_ref[...], b_ref[...]).astype(jnp.bfloat16)

    dstv = dst_ref[0]
    d_lo = jnp.min(dstv)
    d_hi = jnp.max(dstv)
    start = (d_lo // 8) * 8
    nchunk = (d_hi - start) // SEG_W + 1

    def chunk(c, _):
        row0 = start + c * SEG_W
        rows = lax.broadcasted_iota(jnp.int32, (SEG_W, BE), 0) + row0
        oh = (rows == dstv).astype(jnp.bfloat16)
        part = jnp.dot(oh, ef, preferred_element_type=jnp.float32)
        agg_ref[pl.ds(row0, SEG_W), :] += part
        return 0

    lax.fori_loop(0, nchunk, chunk, 0)


def _edge_agg(g2m, ag, bg, dst3, w1a, b1, w2, b2, g, b):
    vspec = lambda r, c: pl.BlockSpec((r, c), lambda i: (0, 0))
    return pl.pallas_call(
        _edge_agg_body,
        grid=(NEB,),
        in_specs=[
            pl.BlockSpec((BE, D), lambda i: (i, 0)),
            pl.BlockSpec((BE, H), lambda i: (i, 0)),
            pl.BlockSpec((BE, H), lambda i: (i, 0)),
            pl.BlockSpec((1, 1, BE), lambda i: (i, 0, 0)),
            vspec(D, H), vspec(1, H), vspec(H, D), vspec(1, D),
            vspec(1, D), vspec(1, D),
        ],
        out_specs=pl.BlockSpec((MPAD, D), lambda i: (0, 0)),
        out_shape=jax.ShapeDtypeStruct((MPAD, D), jnp.float32),
    )(g2m, ag, bg, dst3, w1a, b1, w2, b2, g, b)


def _node_body(x_ref, w1_ref, b1_ref, w2_ref, b2_ref, g_ref, b_ref, o_ref):
    x = x_ref[...]
    h = jnp.dot(x.astype(jnp.bfloat16), w1_ref[...],
                preferred_element_type=jnp.float32) + b1_ref[...]
    h = h * jax.nn.sigmoid(h)
    y = jnp.dot(h.astype(jnp.bfloat16), w2_ref[...],
                preferred_element_type=jnp.float32) + b2_ref[...]
    o_ref[...] = x + _ln(y, g_ref[...], b_ref[...])


def _node_mlp(x, w1, b1, w2, b2, g, b, blk):
    n = x.shape[0]
    vspec = lambda r, c: pl.BlockSpec((r, c), lambda i: (0, 0))
    return pl.pallas_call(
        _node_body,
        grid=(n // blk,),
        in_specs=[
            pl.BlockSpec((blk, D), lambda i: (i, 0)),
            vspec(D, H), vspec(1, H), vspec(H, D), vspec(1, D),
            vspec(1, D), vspec(1, D),
        ],
        out_specs=pl.BlockSpec((blk, D), lambda i: (i, 0)),
        out_shape=jax.ShapeDtypeStruct((n, D), jnp.float32),
    )(x, w1, b1, w2, b2, g, b)


def _mesh_body(x_ref, agg_ref, w1a_ref, w1b_ref, b1_ref, w2_ref, b2_ref,
               g_ref, b_ref, o_ref):
    x = x_ref[...]
    h = (jnp.dot(x.astype(jnp.bfloat16), w1a_ref[...],
                 preferred_element_type=jnp.float32)
         + jnp.dot(agg_ref[...].astype(jnp.bfloat16), w1b_ref[...],
                   preferred_element_type=jnp.float32)
         + b1_ref[...])
    h = h * jax.nn.sigmoid(h)
    y = jnp.dot(h.astype(jnp.bfloat16), w2_ref[...],
                preferred_element_type=jnp.float32) + b2_ref[...]
    o_ref[...] = x + _ln(y, g_ref[...], b_ref[...])


def _mesh_mlp(xpad, agg, w1a, w1b, b1, w2, b2, g, b):
    vspec = lambda r, c: pl.BlockSpec((r, c), lambda i: (0, 0))
    return pl.pallas_call(
        _mesh_body,
        grid=(1,),
        in_specs=[
            vspec(MPAD, D), vspec(MPAD, D),
            vspec(D, H), vspec(D, H), vspec(1, H), vspec(H, D), vspec(1, D),
            vspec(1, D), vspec(1, D),
        ],
        out_specs=pl.BlockSpec((MPAD, D), lambda i: (0, 0)),
        out_shape=jax.ShapeDtypeStruct((MPAD, D), jnp.float32),
    )(xpad, agg, w1a, w1b, b1, w2, b2, g, b)


def kernel(g2m_efeat, grid_nfeat, mesh_nfeat, src_idx, dst_idx,
           W1e, b1e, W2e, b2e, ge, be,
           W1d, b1d, W2d, b2d, gd, bd,
           W1s, b1s, W2s, b2s, gs, bs):
    r1 = lambda v: v.reshape(1, -1)
    bf = lambda w: w.astype(jnp.bfloat16)

    tab_a = _rowmm(grid_nfeat, bf(W1e[D:2 * D]), 2000)
    tab_b = _rowmm(mesh_nfeat, bf(W1e[2 * D:]), 2500)

    ag, bg = _sc_gather(tab_a, tab_b, src_idx, dst_idx)

    dst3 = dst_idx.reshape(NEB, 1, BE)
    agg = _edge_agg(g2m_efeat, ag, bg, dst3, bf(W1e[:D]), r1(b1e), bf(W2e),
                    r1(b2e), r1(ge), r1(be))

    xpad = jnp.pad(mesh_nfeat, ((0, MPAD - NM), (0, 0)))
    mesh_out = _mesh_mlp(xpad, agg, bf(W1d[:D]), bf(W1d[D:]), r1(b1d),
                         bf(W2d), r1(b2d), r1(gd), r1(bd))[:NM]
    grid_out = _node_mlp(grid_nfeat, bf(W1s), r1(b1s), bf(W2s), r1(b2s),
                         r1(gs), r1(bs), 2000)
    return (grid_out, mesh_out)


# speedup vs baseline: 4.6421x; 2.3274x over previous
import functools

import jax
import jax.numpy as jnp
from jax import lax
from jax.experimental import pallas as pl
from jax.experimental.pallas import tpu as pltpu
from jax.experimental.pallas import tpu_sc as plsc

NG = 10000
NM = 2500
NE = 160000
D = 256
H = 256
HH = H // 2

SEG_W = 128
MPAD = 2496 + SEG_W

NC = 2
NS = 16
NW = NC * NS

GBLK = 200
PER_W = NE // NW
NBLK = PER_W // GBLK

BE = 2000
NEB = NE // BE

_MASK_HI = -65536


def _sc_gather(tab, src_idx):
    @functools.partial(
        pl.kernel,
        mesh=plsc.VectorSubcoreMesh(core_axis_name="c", subcore_axis_name="s"),
        out_type=jax.ShapeDtypeStruct((NE, HH), jnp.float32),
        scratch_types=[
            pltpu.VMEM((GBLK,), jnp.int32),
            pltpu.VMEM((GBLK, HH), jnp.float32),
            pltpu.SemaphoreType.DMA,
        ],
    )
    def k(a_hbm, si_hbm, oa_hbm, idx_v, rows_v, sem):
        wid = lax.axis_index("s") * NC + lax.axis_index("c")
        base = wid * PER_W

        @pl.loop(0, NBLK)
        def _(i):
            off = base + i * GBLK
            pltpu.sync_copy(si_hbm.at[pl.ds(off, GBLK)], idx_v)
            pltpu.async_copy(a_hbm.at[idx_v], rows_v, sem).wait()
            pltpu.sync_copy(rows_v, oa_hbm.at[pl.ds(off, GBLK)])

    return k(tab, src_idx)


def _bf(x):
    return x.astype(jnp.bfloat16)


def _mm_body(x_ref, w_ref, o_ref):
    o_ref[...] = jnp.dot(_bf(x_ref[...]), w_ref[...],
                         preferred_element_type=jnp.float32)


def _pack_body(x_ref, w_ref, o_ref):
    y = jnp.dot(_bf(x_ref[...]), w_ref[...],
                preferred_element_type=jnp.float32)
    ilo = lax.bitcast_convert_type(_bf(y[:, :HH]).astype(jnp.float32),
                                   jnp.int32)
    ihi = lax.bitcast_convert_type(_bf(y[:, HH:]).astype(jnp.float32),
                                   jnp.int32)
    packed = (ihi & _MASK_HI) | lax.shift_right_logical(ilo, 16)
    o_ref[...] = lax.bitcast_convert_type(packed, jnp.float32)


def _rowmm(x, w, blk, body, out_cols):
    n = x.shape[0]
    return pl.pallas_call(
        body,
        grid=(n // blk,),
        in_specs=[
            pl.BlockSpec((blk, x.shape[1]), lambda i: (i, 0)),
            pl.BlockSpec((w.shape[0], w.shape[1]), lambda i: (0, 0)),
        ],
        out_specs=pl.BlockSpec((blk, out_cols), lambda i: (i, 0)),
        out_shape=jax.ShapeDtypeStruct((n, out_cols), jnp.float32),
    )(x, w)


def _ln(y, g, b):
    mu = jnp.mean(y, axis=-1, keepdims=True)
    d = y - mu
    var = jnp.mean(d * d, axis=-1, keepdims=True)
    return d * lax.rsqrt(var + 1e-5) * g + b


def _edge_agg_body(x_ref, agp_ref, dst_ref, tabb_ref, w1_ref, b1_ref,
                   w2_ref, b2_ref, g_ref, b_ref, agg_ref, bg_ref):
    i = pl.program_id(0)

    @pl.when(i == 0)
    def _():
        agg_ref[...] = jnp.zeros_like(agg_ref)

    dstv = dst_ref[0]
    d_lo = jnp.min(dstv)
    d_hi = jnp.max(dstv)
    start = (d_lo // 8) * 8
    nchunk = (d_hi - start) // SEG_W + 1

    bg_ref[...] = jnp.zeros_like(bg_ref)

    def exp_chunk(c, _):
        row0 = start + c * SEG_W
        rows = lax.broadcasted_iota(jnp.int32, (SEG_W, BE), 0) + row0
        oh = _bf(rows == dstv)
        band = _bf(tabb_ref[pl.ds(row0, SEG_W), :])
        bg_ref[...] += lax.dot_general(
            oh, band, (((0,), (0,)), ((), ())),
            preferred_element_type=jnp.float32)
        return 0

    lax.fori_loop(0, nchunk, exp_chunk, 0)

    w = lax.bitcast_convert_type(agp_ref[...], jnp.int32)
    ag_lo = lax.bitcast_convert_type(lax.shift_left(w, 16), jnp.float32)
    ag_hi = lax.bitcast_convert_type(w & _MASK_HI, jnp.float32)

    bg = bg_ref[...]
    xb = _bf(x_ref[...])
    h1 = (jnp.dot(xb, w1_ref[:, :HH], preferred_element_type=jnp.float32)
          + ag_lo + bg[:, :HH] + b1_ref[:, :HH])
    h2 = (jnp.dot(xb, w1_ref[:, HH:], preferred_element_type=jnp.float32)
          + ag_hi + bg[:, HH:] + b1_ref[:, HH:])
    h1 = h1 * jax.nn.sigmoid(h1)
    h2 = h2 * jax.nn.sigmoid(h2)
    y = (jnp.dot(_bf(h1), w2_ref[:HH, :], preferred_element_type=jnp.float32)
         + jnp.dot(_bf(h2), w2_ref[HH:, :], preferred_element_type=jnp.float32)
         + b2_ref[...])
    ef = _bf(_ln(y, g_ref[...], b_ref[...]))

    def agg_chunk(c, _):
        row0 = start + c * SEG_W
        rows = lax.broadcasted_iota(jnp.int32, (SEG_W, BE), 0) + row0
        oh = _bf(rows == dstv)
        part = jnp.dot(oh, ef, preferred_element_type=jnp.float32)
        agg_ref[pl.ds(row0, SEG_W), :] += part
        return 0

    lax.fori_loop(0, nchunk, agg_chunk, 0)


def _edge_agg(g2m, agp, dst3, tabb, w1, b1, w2, b2, g, b):
    vspec = lambda r, c: pl.BlockSpec((r, c), lambda i: (0, 0))
    return pl.pallas_call(
        _edge_agg_body,
        grid=(NEB,),
        in_specs=[
            pl.BlockSpec((BE, D), lambda i: (i, 0)),
            pl.BlockSpec((BE, HH), lambda i: (i, 0)),
            pl.BlockSpec((1, 1, BE), lambda i: (i, 0, 0)),
            vspec(MPAD, D),
            vspec(D, H), vspec(1, H), vspec(H, D), vspec(1, D),
            vspec(1, D), vspec(1, D),
        ],
        out_specs=pl.BlockSpec((MPAD, D), lambda i: (0, 0)),
        out_shape=jax.ShapeDtypeStruct((MPAD, D), jnp.float32),
        scratch_shapes=[pltpu.VMEM((BE, D), jnp.float32)],
    )(g2m, agp, dst3, tabb, w1, b1, w2, b2, g, b)


def _node_body(x_ref, w1_ref, b1_ref, w2_ref, b2_ref, g_ref, b_ref, o_ref):
    x = x_ref[...]
    h = jnp.dot(_bf(x), w1_ref[...],
                preferred_element_type=jnp.float32) + b1_ref[...]
    h = h * jax.nn.sigmoid(h)
    y = jnp.dot(_bf(h), w2_ref[...],
                preferred_element_type=jnp.float32) + b2_ref[...]
    o_ref[...] = x + _ln(y, g_ref[...], b_ref[...])


def _node_mlp(x, w1, b1, w2, b2, g, b, blk):
    n = x.shape[0]
    vspec = lambda r, c: pl.BlockSpec((r, c), lambda i: (0, 0))
    return pl.pallas_call(
        _node_body,
        grid=(n // blk,),
        in_specs=[
            pl.BlockSpec((blk, D), lambda i: (i, 0)),
            vspec(D, H), vspec(1, H), vspec(H, D), vspec(1, D),
            vspec(1, D), vspec(1, D),
        ],
        out_specs=pl.BlockSpec((blk, D), lambda i: (i, 0)),
        out_shape=jax.ShapeDtypeStruct((n, D), jnp.float32),
    )(x, w1, b1, w2, b2, g, b)


def _mesh_body(x_ref, agg_ref, w1a_ref, w1b_ref, b1_ref, w2_ref, b2_ref,
               g_ref, b_ref, o_ref):
    x = x_ref[...]
    h = (jnp.dot(_bf(x), w1a_ref[...], preferred_element_type=jnp.float32)
         + jnp.dot(_bf(agg_ref[...]), w1b_ref[...],
                   preferred_element_type=jnp.float32)
         + b1_ref[...])
    h = h * jax.nn.sigmoid(h)
    y = jnp.dot(_bf(h), w2_ref[...],
                preferred_element_type=jnp.float32) + b2_ref[...]
    o_ref[...] = x + _ln(y, g_ref[...], b_ref[...])


def _mesh_mlp(xpad, agg, w1a, w1b, b1, w2, b2, g, b):
    vspec = lambda r, c: pl.BlockSpec((r, c), lambda i: (0, 0))
    return pl.pallas_call(
        _mesh_body,
        grid=(1,),
        in_specs=[
            vspec(MPAD, D), vspec(MPAD, D),
            vspec(D, H), vspec(D, H), vspec(1, H), vspec(H, D), vspec(1, D),
            vspec(1, D), vspec(1, D),
        ],
        out_specs=pl.BlockSpec((MPAD, D), lambda i: (0, 0)),
        out_shape=jax.ShapeDtypeStruct((MPAD, D), jnp.float32),
    )(xpad, agg, w1a, w1b, b1, w2, b2, g, b)


def kernel(g2m_efeat, grid_nfeat, mesh_nfeat, src_idx, dst_idx,
           W1e, b1e, W2e, b2e, ge, be,
           W1d, b1d, W2d, b2d, gd, bd,
           W1s, b1s, W2s, b2s, gs, bs):
    r1 = lambda v: v.reshape(1, -1)
    bf = lambda w: w.astype(jnp.bfloat16)

    xpad = jnp.pad(mesh_nfeat, ((0, MPAD - NM), (0, 0)))

    tab_a = _rowmm(grid_nfeat, bf(W1e[D:2 * D]), 2000, _pack_body, HH)
    tab_b = _rowmm(xpad, bf(W1e[2 * D:]), MPAD, _mm_body, D)

    agp = _sc_gather(tab_a, src_idx)

    dst3 = dst_idx.reshape(NEB, 1, BE)
    agg = _edge_agg(g2m_efeat, agp, dst3, tab_b, bf(W1e[:D]), r1(b1e),
                    bf(W2e), r1(b2e), r1(ge), r1(be))

    mesh_out = _mesh_mlp(xpad, agg, bf(W1d[:D]), bf(W1d[D:]), r1(b1d),
                         bf(W2d), r1(b2d), r1(gd), r1(bd))[:NM]
    grid_out = _node_mlp(grid_nfeat, bf(W1s), r1(b1s), bf(W2s), r1(b2s),
                         r1(gs), r1(bs), 2000)
    return (grid_out, mesh_out)


# speedup vs baseline: 6.0965x; 1.3133x over previous
import functools

import jax
import jax.numpy as jnp
from jax import lax
from jax.experimental import pallas as pl
from jax.experimental.pallas import tpu as pltpu
from jax.experimental.pallas import tpu_sc as plsc

NG = 10000
NM = 2500
NE = 160000
D = 256
H = 256
HH = H // 2

SEG_W = 64
MPAD = 2496 + SEG_W

NC = 2
NS = 16
NW = NC * NS

GBLK = 200

BE = 2000
EH0 = 64000
EH1 = NE - EH0

_MASK_HI = -65536


def _sc_gather(tab, src_idx, off, n_edges):
    per_w = n_edges // NW
    nblk = per_w // GBLK

    @functools.partial(
        pl.kernel,
        mesh=plsc.VectorSubcoreMesh(core_axis_name="c", subcore_axis_name="s"),
        out_type=jax.ShapeDtypeStruct((n_edges, HH), jnp.float32),
        scratch_types=[
            pltpu.VMEM((GBLK,), jnp.int32),
            pltpu.VMEM((GBLK,), jnp.int32),
            pltpu.VMEM((GBLK, HH), jnp.float32),
            pltpu.VMEM((GBLK, HH), jnp.float32),
            pltpu.SemaphoreType.DMA,
            pltpu.SemaphoreType.DMA,
        ],
    )
    def k(a_hbm, si_hbm, oa_hbm, idx0, idx1, r0, r1, sg0, sg1):
        wid = lax.axis_index("s") * NC + lax.axis_index("c")
        base = wid * per_w
        pltpu.sync_copy(si_hbm.at[pl.ds(off + base, GBLK)], idx0)
        pltpu.async_copy(a_hbm.at[idx0], r0, sg0)
        pltpu.sync_copy(si_hbm.at[pl.ds(off + base + GBLK, GBLK)], idx1)
        pltpu.async_copy(a_hbm.at[idx1], r1, sg1)

        @pl.loop(0, nblk // 2)
        def _(p):
            o = base + 2 * p * GBLK
            pltpu.make_async_copy(a_hbm.at[idx0], r0, sg0).wait()
            pltpu.sync_copy(r0, oa_hbm.at[pl.ds(o, GBLK)])

            @pl.when(2 * p + 2 < nblk)
            def _():
                pltpu.sync_copy(si_hbm.at[pl.ds(off + o + 2 * GBLK, GBLK)],
                                idx0)
                pltpu.async_copy(a_hbm.at[idx0], r0, sg0)

            pltpu.make_async_copy(a_hbm.at[idx1], r1, sg1).wait()
            pltpu.sync_copy(r1, oa_hbm.at[pl.ds(o + GBLK, GBLK)])

            @pl.when(2 * p + 3 < nblk)
            def _():
                pltpu.sync_copy(si_hbm.at[pl.ds(off + o + 3 * GBLK, GBLK)],
                                idx1)
                pltpu.async_copy(a_hbm.at[idx1], r1, sg1)

        if nblk % 2:
            o = base + (nblk - 1) * GBLK
            pltpu.make_async_copy(a_hbm.at[idx0], r0, sg0).wait()
            pltpu.sync_copy(r0, oa_hbm.at[pl.ds(o, GBLK)])

    return k(tab, src_idx)


def _bf(x):
    return x.astype(jnp.bfloat16)


def _mm_body(x_ref, w_ref, o_ref):
    o_ref[...] = jnp.dot(_bf(x_ref[...]), w_ref[...],
                         preferred_element_type=jnp.float32)


def _mm_bias_body(x_ref, w_ref, b_ref, o_ref):
    o_ref[...] = jnp.dot(_bf(x_ref[...]), w_ref[...],
                         preferred_element_type=jnp.float32) + b_ref[...]


def _pack_body(x_ref, w_ref, o_ref):
    y = jnp.dot(_bf(x_ref[...]), w_ref[...],
                preferred_element_type=jnp.float32)
    ilo = lax.bitcast_convert_type(_bf(y[:, :HH]).astype(jnp.float32),
                                   jnp.int32)
    ihi = lax.bitcast_convert_type(_bf(y[:, HH:]).astype(jnp.float32),
                                   jnp.int32)
    packed = (ihi & _MASK_HI) | lax.shift_right_logical(ilo, 16)
    o_ref[...] = lax.bitcast_convert_type(packed, jnp.float32)


def _rowmm(x, w, blk, body, out_cols, *extra):
    n = x.shape[0]
    return pl.pallas_call(
        body,
        grid=(n // blk,),
        in_specs=[
            pl.BlockSpec((blk, x.shape[1]), lambda i: (i, 0)),
            pl.BlockSpec((w.shape[0], w.shape[1]), lambda i: (0, 0)),
        ] + [pl.BlockSpec(e.shape, lambda i: (0, 0)) for e in extra],
        out_specs=pl.BlockSpec((blk, out_cols), lambda i: (i, 0)),
        out_shape=jax.ShapeDtypeStruct((n, out_cols), jnp.float32),
    )(x, w, *extra)


def _ln(y, g, b):
    mu = jnp.mean(y, axis=-1, keepdims=True)
    d = y - mu
    var = jnp.mean(d * d, axis=-1, keepdims=True)
    return d * lax.rsqrt(var + 1e-5) * g + b


def _edge_agg_body(x_ref, agp_ref, dst_ref, tabb_ref, w1_ref,
                   w2_ref, b2_ref, g_ref, b_ref, agg_ref, bg_ref):
    i = pl.program_id(0)

    @pl.when(i == 0)
    def _():
        agg_ref[...] = jnp.zeros_like(agg_ref)

    dstv = dst_ref[0]
    d_lo = jnp.min(dstv)
    d_hi = jnp.max(dstv)
    start = (d_lo // 8) * 8
    nchunk = (d_hi - start) // SEG_W + 1

    def exp_chunk(c, _):
        row0 = start + c * SEG_W
        rows = lax.broadcasted_iota(jnp.int32, (SEG_W, BE), 0) + row0
        oh = _bf(rows == dstv)
        band = _bf(tabb_ref[pl.ds(row0, SEG_W), :])
        contrib = lax.dot_general(oh, band, (((0,), (0,)), ((), ())),
                                  preferred_element_type=jnp.float32)

        @pl.when(c == 0)
        def _():
            bg_ref[...] = contrib

        @pl.when(c > 0)
        def _():
            bg_ref[...] += contrib

        return 0

    lax.fori_loop(0, nchunk, exp_chunk, 0)

    w = lax.bitcast_convert_type(agp_ref[...], jnp.int32)
    ag = jnp.concatenate(
        [lax.bitcast_convert_type(lax.shift_left(w, 16), jnp.float32),
         lax.bitcast_convert_type(w & _MASK_HI, jnp.float32)], axis=1)

    xb = _bf(x_ref[...])
    h = (jnp.dot(xb, w1_ref[...], preferred_element_type=jnp.float32)
         + ag + bg_ref[...])
    h = _bf(h)
    h = h * jax.nn.sigmoid(h)
    y = jnp.dot(h, w2_ref[...], preferred_element_type=jnp.float32) + b2_ref[...]
    ef = _bf(_ln(y, g_ref[...], b_ref[...]))

    def agg_chunk(c, _):
        row0 = start + c * SEG_W
        rows = lax.broadcasted_iota(jnp.int32, (SEG_W, BE), 0) + row0
        oh = _bf(rows == dstv)
        part = jnp.dot(oh, ef, preferred_element_type=jnp.float32)
        agg_ref[pl.ds(row0, SEG_W), :] += part
        return 0

    lax.fori_loop(0, nchunk, agg_chunk, 0)


def _edge_agg(g2m, agp, dst3, tabb, w1, w2, b2, g, b, off_blk, nb):
    vspec = lambda r, c: pl.BlockSpec((r, c), lambda i: (0, 0))
    return pl.pallas_call(
        _edge_agg_body,
        grid=(nb,),
        in_specs=[
            pl.BlockSpec((BE, D), lambda i: (i + off_blk, 0)),
            pl.BlockSpec((BE, HH), lambda i: (i, 0)),
            pl.BlockSpec((1, 1, BE), lambda i: (i + off_blk, 0, 0)),
            vspec(MPAD, D),
            vspec(D, H), vspec(H, D), vspec(1, D),
            vspec(1, D), vspec(1, D),
        ],
        out_specs=pl.BlockSpec((MPAD, D), lambda i: (0, 0)),
        out_shape=jax.ShapeDtypeStruct((MPAD, D), jnp.float32),
        scratch_shapes=[pltpu.VMEM((BE, D), jnp.float32)],
    )(g2m, agp, dst3, tabb, w1, w2, b2, g, b)


def _node_body(x_ref, w1_ref, b1_ref, w2_ref, b2_ref, g_ref, b_ref, o_ref):
    x = x_ref[...]
    h = jnp.dot(_bf(x), w1_ref[...],
                preferred_element_type=jnp.float32) + b1_ref[...]
    h = _bf(h)
    h = h * jax.nn.sigmoid(h)
    y = jnp.dot(h, w2_ref[...],
                preferred_element_type=jnp.float32) + b2_ref[...]
    o_ref[...] = x + _ln(y, g_ref[...], b_ref[...])


def _node_mlp(x, w1, b1, w2, b2, g, b, blk):
    n = x.shape[0]
    vspec = lambda r, c: pl.BlockSpec((r, c), lambda i: (0, 0))
    return pl.pallas_call(
        _node_body,
        grid=(n // blk,),
        in_specs=[
            pl.BlockSpec((blk, D), lambda i: (i, 0)),
            vspec(D, H), vspec(1, H), vspec(H, D), vspec(1, D),
            vspec(1, D), vspec(1, D),
        ],
        out_specs=pl.BlockSpec((blk, D), lambda i: (i, 0)),
        out_shape=jax.ShapeDtypeStruct((n, D), jnp.float32),
    )(x, w1, b1, w2, b2, g, b)


def _mesh_body(x_ref, a0_ref, a1_ref, w1a_ref, w1b_ref, b1_ref, w2_ref,
               b2_ref, g_ref, b_ref, o_ref):
    x = x_ref[...]
    agg = a0_ref[...] + a1_ref[...]
    h = (jnp.dot(_bf(x), w1a_ref[...], preferred_element_type=jnp.float32)
         + jnp.dot(_bf(agg), w1b_ref[...], preferred_element_type=jnp.float32)
         + b1_ref[...])
    h = _bf(h)
    h = h * jax.nn.sigmoid(h)
    y = jnp.dot(h, w2_ref[...],
                preferred_element_type=jnp.float32) + b2_ref[...]
    o_ref[...] = x + _ln(y, g_ref[...], b_ref[...])


def _mesh_mlp(xpad, a0, a1, w1a, w1b, b1, w2, b2, g, b):
    vspec = lambda r, c: pl.BlockSpec((r, c), lambda i: (0, 0))
    return pl.pallas_call(
        _mesh_body,
        grid=(1,),
        in_specs=[
            vspec(MPAD, D), vspec(MPAD, D), vspec(MPAD, D),
            vspec(D, H), vspec(D, H), vspec(1, H), vspec(H, D), vspec(1, D),
            vspec(1, D), vspec(1, D),
        ],
        out_specs=pl.BlockSpec((MPAD, D), lambda i: (0, 0)),
        out_shape=jax.ShapeDtypeStruct((MPAD, D), jnp.float32),
    )(xpad, a0, a1, w1a, w1b, b1, w2, b2, g, b)


def kernel(g2m_efeat, grid_nfeat, mesh_nfeat, src_idx, dst_idx,
           W1e, b1e, W2e, b2e, ge, be,
           W1d, b1d, W2d, b2d, gd, bd,
           W1s, b1s, W2s, b2s, gs, bs):
    r1 = lambda v: v.reshape(1, -1)
    bf = lambda w: w.astype(jnp.bfloat16)

    xpad = jnp.pad(mesh_nfeat, ((0, MPAD - NM), (0, 0)))

    tab_a = _rowmm(grid_nfeat, bf(W1e[D:2 * D]), 2000, _pack_body, HH)
    tab_b = _rowmm(xpad, bf(W1e[2 * D:]), MPAD, _mm_bias_body, D, r1(b1e))

    agp0 = _sc_gather(tab_a, src_idx, 0, EH0)
    agp1 = _sc_gather(tab_a, src_idx, EH0, EH1)

    dst3 = dst_idx.reshape(NE // BE, 1, BE)
    w1a, w2, b2r = bf(W1e[:D]), bf(W2e), r1(b2e)
    ger, ber = r1(ge), r1(be)
    agg0 = _edge_agg(g2m_efeat, agp0, dst3, tab_b, w1a, w2, b2r,
                     ger, ber, 0, EH0 // BE)
    agg1 = _edge_agg(g2m_efeat, agp1, dst3, tab_b, w1a, w2, b2r,
                     ger, ber, EH0 // BE, EH1 // BE)

    mesh_out = _mesh_mlp(xpad, agg0, agg1, bf(W1d[:D]), bf(W1d[D:]), r1(b1d),
                         bf(W2d), r1(b2d), r1(gd), r1(bd))[:NM]
    grid_out = _node_mlp(grid_nfeat, bf(W1s), r1(b1s), bf(W2s), r1(b2s),
                         r1(gs), r1(bs), 2000)
    return (grid_out, mesh_out)


# speedup vs baseline: 6.1714x; 1.0123x over previous
import functools

import jax
import jax.numpy as jnp
from jax import lax
from jax.experimental import pallas as pl
from jax.experimental.pallas import tpu as pltpu
from jax.experimental.pallas import tpu_sc as plsc

NG = 10000
NM = 2500
NE = 160000
D = 256
H = 256
HH = H // 2

SEG_W = 48
MPAD = 2496 + SEG_W

NC = 2
NS = 16
NW = NC * NS

GBLK = 200

BE = 2000
EH0 = 64000
EH1 = NE - EH0

_MASK_HI = -65536


def _sc_gather(tab, src_idx, off, n_edges):
    per_w = n_edges // NW
    nblk = per_w // GBLK

    @functools.partial(
        pl.kernel,
        mesh=plsc.VectorSubcoreMesh(core_axis_name="c", subcore_axis_name="s"),
        out_type=jax.ShapeDtypeStruct((n_edges, HH), jnp.float32),
        scratch_types=[
            pltpu.VMEM((GBLK,), jnp.int32),
            pltpu.VMEM((GBLK,), jnp.int32),
            pltpu.VMEM((GBLK, HH), jnp.float32),
            pltpu.VMEM((GBLK, HH), jnp.float32),
            pltpu.SemaphoreType.DMA,
            pltpu.SemaphoreType.DMA,
        ],
    )
    def k(a_hbm, si_hbm, oa_hbm, idx0, idx1, r0, r1, sg0, sg1):
        wid = lax.axis_index("s") * NC + lax.axis_index("c")
        base = wid * per_w
        pltpu.sync_copy(si_hbm.at[pl.ds(off + base, GBLK)], idx0)
        pltpu.async_copy(a_hbm.at[idx0], r0, sg0)
        pltpu.sync_copy(si_hbm.at[pl.ds(off + base + GBLK, GBLK)], idx1)
        pltpu.async_copy(a_hbm.at[idx1], r1, sg1)

        @pl.loop(0, nblk // 2)
        def _(p):
            o = base + 2 * p * GBLK
            pltpu.make_async_copy(a_hbm.at[idx0], r0, sg0).wait()
            pltpu.sync_copy(r0, oa_hbm.at[pl.ds(o, GBLK)])

            @pl.when(2 * p + 2 < nblk)
            def _():
                pltpu.sync_copy(si_hbm.at[pl.ds(off + o + 2 * GBLK, GBLK)],
                                idx0)
                pltpu.async_copy(a_hbm.at[idx0], r0, sg0)

            pltpu.make_async_copy(a_hbm.at[idx1], r1, sg1).wait()
            pltpu.sync_copy(r1, oa_hbm.at[pl.ds(o + GBLK, GBLK)])

            @pl.when(2 * p + 3 < nblk)
            def _():
                pltpu.sync_copy(si_hbm.at[pl.ds(off + o + 3 * GBLK, GBLK)],
                                idx1)
                pltpu.async_copy(a_hbm.at[idx1], r1, sg1)

        if nblk % 2:
            o = base + (nblk - 1) * GBLK
            pltpu.make_async_copy(a_hbm.at[idx0], r0, sg0).wait()
            pltpu.sync_copy(r0, oa_hbm.at[pl.ds(o, GBLK)])

    return k(tab, src_idx)


def _bf(x):
    return x.astype(jnp.bfloat16)


def _ln(y, g, b):
    mu = jnp.mean(y, axis=-1, keepdims=True)
    d = y - mu
    var = jnp.mean(d * d, axis=-1, keepdims=True)
    return d * lax.rsqrt(var + 1e-5) * g + b


def _pregrid_body(x_ref, wm_ref, w1_ref, b1_ref, w2_ref, b2_ref, g_ref,
                  b_ref, tab_ref, go_ref):
    x = x_ref[...]
    xb = _bf(x)
    y = jnp.dot(xb, wm_ref[...], preferred_element_type=jnp.float32)
    ilo = lax.bitcast_convert_type(_bf(y[:, :HH]).astype(jnp.float32),
                                   jnp.int32)
    ihi = lax.bitcast_convert_type(_bf(y[:, HH:]).astype(jnp.float32),
                                   jnp.int32)
    packed = (ihi & _MASK_HI) | lax.shift_right_logical(ilo, 16)
    tab_ref[...] = lax.bitcast_convert_type(packed, jnp.float32)

    h = jnp.dot(xb, w1_ref[...], preferred_element_type=jnp.float32) + b1_ref[...]
    h = _bf(h)
    h = h * jax.nn.sigmoid(h)
    z = jnp.dot(h, w2_ref[...], preferred_element_type=jnp.float32) + b2_ref[...]
    go_ref[...] = x + _ln(z, g_ref[...], b_ref[...])


def _pregrid(x, wm, w1, b1, w2, b2, g, b):
    blk = 2000
    vspec = lambda r, c: pl.BlockSpec((r, c), lambda i: (0, 0))
    return pl.pallas_call(
        _pregrid_body,
        grid=(NG // blk,),
        in_specs=[
            pl.BlockSpec((blk, D), lambda i: (i, 0)),
            vspec(D, H), vspec(D, H), vspec(1, H), vspec(H, D), vspec(1, D),
            vspec(1, D), vspec(1, D),
        ],
        out_specs=(pl.BlockSpec((blk, HH), lambda i: (i, 0)),
                   pl.BlockSpec((blk, D), lambda i: (i, 0))),
        out_shape=(jax.ShapeDtypeStruct((NG, HH), jnp.float32),
                   jax.ShapeDtypeStruct((NG, D), jnp.float32)),
    )(x, wm, w1, b1, w2, b2, g, b)


def _edge_core(x_ref, agp_ref, dst_ref, xp_ref, w1c_ref, b1e_ref, w1_ref,
               w2_ref, b2_ref, g_ref, b_ref, agg_ref, bg_ref, tabb_ref, nb):
    i = pl.program_id(0)

    @pl.when(i == 0)
    def _():
        agg_ref[...] = jnp.zeros_like(agg_ref)
        tabb_ref[...] = (jnp.dot(_bf(xp_ref[...]), w1c_ref[...],
                                 preferred_element_type=jnp.float32)
                         + b1e_ref[...])

    dstv = dst_ref[0]
    d_lo = jnp.min(dstv)
    d_hi = jnp.max(dstv)
    start = (d_lo // 8) * 8
    nchunk = (d_hi - start) // SEG_W + 1

    def exp_chunk(c, _):
        row0 = start + c * SEG_W
        rows = lax.broadcasted_iota(jnp.int32, (SEG_W, BE), 0) + row0
        oh = _bf(rows == dstv)
        band = _bf(tabb_ref[pl.ds(row0, SEG_W), :])
        contrib = lax.dot_general(oh, band, (((0,), (0,)), ((), ())),
                                  preferred_element_type=jnp.float32)

        @pl.when(c == 0)
        def _():
            bg_ref[...] = contrib

        @pl.when(c > 0)
        def _():
            bg_ref[...] += contrib

        return 0

    lax.fori_loop(0, nchunk, exp_chunk, 0)

    w = lax.bitcast_convert_type(agp_ref[...], jnp.int32)
    ag = jnp.concatenate(
        [lax.bitcast_convert_type(lax.shift_left(w, 16), jnp.float32),
         lax.bitcast_convert_type(w & _MASK_HI, jnp.float32)], axis=1)

    xb = _bf(x_ref[...])
    h = (jnp.dot(xb, w1_ref[...], preferred_element_type=jnp.float32)
         + ag + bg_ref[...])
    h = _bf(h)
    h = h * jax.nn.sigmoid(h)
    y = jnp.dot(h, w2_ref[...], preferred_element_type=jnp.float32) + b2_ref[...]
    ef = _bf(_ln(y, g_ref[...], b_ref[...]))

    def agg_chunk(c, _):
        row0 = start + c * SEG_W
        rows = lax.broadcasted_iota(jnp.int32, (SEG_W, BE), 0) + row0
        oh = _bf(rows == dstv)
        part = jnp.dot(oh, ef, preferred_element_type=jnp.float32)
        agg_ref[pl.ds(row0, SEG_W), :] += part
        return 0

    lax.fori_loop(0, nchunk, agg_chunk, 0)


def _edge0_body(x_ref, agp_ref, dst_ref, xp_ref, w1c_ref, b1e_ref, w1_ref,
                w2_ref, b2_ref, g_ref, b_ref, agg_ref, bg_ref, tabb_ref):
    _edge_core(x_ref, agp_ref, dst_ref, xp_ref, w1c_ref, b1e_ref, w1_ref,
               w2_ref, b2_ref, g_ref, b_ref, agg_ref, bg_ref, tabb_ref,
               EH0 // BE)


def _edge1_body(x_ref, agp_ref, dst_ref, xp_ref, w1c_ref, b1e_ref, w1_ref,
                w2_ref, b2_ref, g_ref, b_ref, a0_ref, w1da_ref, w1db_ref,
                b1d_ref, w2d_ref, b2d_ref, gd_ref, bd_ref, mesh_ref,
                bg_ref, tabb_ref, agg_ref):
    nb = EH1 // BE
    _edge_core(x_ref, agp_ref, dst_ref, xp_ref, w1c_ref, b1e_ref, w1_ref,
               w2_ref, b2_ref, g_ref, b_ref, agg_ref, bg_ref, tabb_ref, nb)

    @pl.when(pl.program_id(0) == nb - 1)
    def _():
        x = xp_ref[...]
        agg = agg_ref[...] + a0_ref[...]
        h = (jnp.dot(_bf(x), w1da_ref[...], preferred_element_type=jnp.float32)
             + jnp.dot(_bf(agg), w1db_ref[...],
                       preferred_element_type=jnp.float32)
             + b1d_ref[...])
        h = _bf(h)
        h = h * jax.nn.sigmoid(h)
        y = (jnp.dot(h, w2d_ref[...], preferred_element_type=jnp.float32)
             + b2d_ref[...])
        mesh_ref[...] = x + _ln(y, gd_ref[...], bd_ref[...])


def _edge0(g2m, agp, dst3, xpad, w1c, b1e, w1, w2, b2, g, b):
    vspec = lambda r, c: pl.BlockSpec((r, c), lambda i: (0, 0))
    return pl.pallas_call(
        _edge0_body,
        grid=(EH0 // BE,),
        in_specs=[
            pl.BlockSpec((BE, D), lambda i: (i, 0)),
            pl.BlockSpec((BE, HH), lambda i: (i, 0)),
            pl.BlockSpec((1, 1, BE), lambda i: (i, 0, 0)),
            vspec(MPAD, D), vspec(D, H), vspec(1, H),
            vspec(D, H), vspec(H, D), vspec(1, D), vspec(1, D), vspec(1, D),
        ],
        out_specs=pl.BlockSpec((MPAD, D), lambda i: (0, 0)),
        out_shape=jax.ShapeDtypeStruct((MPAD, D), jnp.float32),
        scratch_shapes=[pltpu.VMEM((BE, D), jnp.float32),
                        pltpu.VMEM((MPAD, D), jnp.float32)],
    )(g2m, agp, dst3, xpad, w1c, b1e, w1, w2, b2, g, b)


def _edge1(g2m, agp, dst3, xpad, w1c, b1e, w1, w2, b2, g, b,
           a0, w1da, w1db, b1d, w2d, b2d, gd, bd):
    vspec = lambda r, c: pl.BlockSpec((r, c), lambda i: (0, 0))
    off = EH0 // BE
    return pl.pallas_call(
        _edge1_body,
        grid=(EH1 // BE,),
        in_specs=[
            pl.BlockSpec((BE, D), lambda i: (i + off, 0)),
            pl.BlockSpec((BE, HH), lambda i: (i, 0)),
            pl.BlockSpec((1, 1, BE), lambda i: (i + off, 0, 0)),
            vspec(MPAD, D), vspec(D, H), vspec(1, H),
            vspec(D, H), vspec(H, D), vspec(1, D), vspec(1, D), vspec(1, D),
            vspec(MPAD, D),
            vspec(D, H), vspec(D, H), vspec(1, H), vspec(H, D), vspec(1, D),
            vspec(1, D), vspec(1, D),
        ],
        out_specs=pl.BlockSpec((MPAD, D), lambda i: (0, 0)),
        out_shape=jax.ShapeDtypeStruct((MPAD, D), jnp.float32),
        scratch_shapes=[pltpu.VMEM((BE, D), jnp.float32),
                        pltpu.VMEM((MPAD, D), jnp.float32),
                        pltpu.VMEM((MPAD, D), jnp.float32)],
    )(g2m, agp, dst3, xpad, w1c, b1e, w1, w2, b2, g, b,
      a0, w1da, w1db, b1d, w2d, b2d, gd, bd)


def kernel(g2m_efeat, grid_nfeat, mesh_nfeat, src_idx, dst_idx,
           W1e, b1e, W2e, b2e, ge, be,
           W1d, b1d, W2d, b2d, gd, bd,
           W1s, b1s, W2s, b2s, gs, bs):
    r1 = lambda v: v.reshape(1, -1)
    bf = lambda w: w.astype(jnp.bfloat16)

    xpad = jnp.pad(mesh_nfeat, ((0, MPAD - NM), (0, 0)))

    tab_a, grid_out = _pregrid(grid_nfeat, bf(W1e[D:2 * D]), bf(W1s),
                               r1(b1s), bf(W2s), r1(b2s), r1(gs), r1(bs))

    agp0 = _sc_gather(tab_a, src_idx, 0, EH0)
    agp1 = _sc_gather(tab_a, src_idx, EH0, EH1)

    dst3 = dst_idx.reshape(NE // BE, 1, BE)
    w1c, b1er = bf(W1e[2 * D:]), r1(b1e)
    w1a, w2, b2r, ger, ber = bf(W1e[:D]), bf(W2e), r1(b2e), r1(ge), r1(be)
    agg0 = _edge0(g2m_efeat, agp0, dst3, xpad, w1c, b1er, w1a, w2, b2r,
                  ger, ber)
    mesh_out = _edge1(g2m_efeat, agp1, dst3, xpad, w1c, b1er, w1a, w2, b2r,
                      ger, ber, agg0, bf(W1d[:D]), bf(W1d[D:]), r1(b1d),
                      bf(W2d), r1(b2d), r1(gd), r1(bd))[:NM]
    return (grid_out, mesh_out)


# speedup vs baseline: 6.2441x; 1.0118x over previous
import functools

import jax
import jax.numpy as jnp
import numpy as np
from jax import lax
from jax.experimental import pallas as pl
from jax.experimental.pallas import tpu as pltpu
from jax.experimental.pallas import tpu_sc as plsc

NG = 10000
NM = 2500
NE = 160000
D = 256
H = 256
HH = H // 2

SEG_W = 48
MPAD = 2496 + SEG_W

NC = 2
NS = 16
NW = NC * NS

GBLK = 200

BE = 2000
EH0 = 64000
EH1 = NE - EH0

_MASK_HI = -65536


def _sc_gather(tab, src_idx, off, n_edges):
    per_w = n_edges // NW
    nblk = per_w // GBLK

    @functools.partial(
        pl.kernel,
        mesh=plsc.VectorSubcoreMesh(core_axis_name="c", subcore_axis_name="s"),
        out_type=jax.ShapeDtypeStruct((n_edges, HH), jnp.float32),
        scratch_types=[
            pltpu.VMEM((GBLK,), jnp.int32),
            pltpu.VMEM((GBLK,), jnp.int32),
            pltpu.VMEM((GBLK, HH), jnp.float32),
            pltpu.VMEM((GBLK, HH), jnp.float32),
            pltpu.SemaphoreType.DMA,
            pltpu.SemaphoreType.DMA,
        ],
    )
    def k(a_hbm, si_hbm, oa_hbm, idx0, idx1, r0, r1, sg0, sg1):
        wid = lax.axis_index("s") * NC + lax.axis_index("c")
        base = wid * per_w
        pltpu.sync_copy(si_hbm.at[pl.ds(off + base, GBLK)], idx0)
        pltpu.async_copy(a_hbm.at[idx0], r0, sg0)
        pltpu.sync_copy(si_hbm.at[pl.ds(off + base + GBLK, GBLK)], idx1)
        pltpu.async_copy(a_hbm.at[idx1], r1, sg1)

        @pl.loop(0, nblk // 2)
        def _(p):
            o = base + 2 * p * GBLK
            pltpu.make_async_copy(a_hbm.at[idx0], r0, sg0).wait()
            pltpu.sync_copy(r0, oa_hbm.at[pl.ds(o, GBLK)])

            @pl.when(2 * p + 2 < nblk)
            def _():
                pltpu.sync_copy(si_hbm.at[pl.ds(off + o + 2 * GBLK, GBLK)],
                                idx0)
                pltpu.async_copy(a_hbm.at[idx0], r0, sg0)

            pltpu.make_async_copy(a_hbm.at[idx1], r1, sg1).wait()
            pltpu.sync_copy(r1, oa_hbm.at[pl.ds(o + GBLK, GBLK)])

            @pl.when(2 * p + 3 < nblk)
            def _():
                pltpu.sync_copy(si_hbm.at[pl.ds(off + o + 3 * GBLK, GBLK)],
                                idx1)
                pltpu.async_copy(a_hbm.at[idx1], r1, sg1)

        if nblk % 2:
            o = base + (nblk - 1) * GBLK
            pltpu.make_async_copy(a_hbm.at[idx0], r0, sg0).wait()
            pltpu.sync_copy(r0, oa_hbm.at[pl.ds(o, GBLK)])

    return k(tab, src_idx)


def _bf(x):
    return x.astype(jnp.bfloat16)


def _ln(y, g, b):
    mu = jnp.mean(y, axis=-1, keepdims=True)
    d = y - mu
    var = jnp.mean(d * d, axis=-1, keepdims=True)
    return d * lax.rsqrt(var + 1e-5) * g + b


def _pregrid_body(x_ref, wm_ref, w1_ref, b1_ref, w2_ref, b2_ref, g_ref,
                  b_ref, tab_ref, go_ref):
    x = x_ref[...]
    xb = _bf(x)
    y = jnp.dot(xb, wm_ref[...], preferred_element_type=jnp.float32)
    ilo = lax.bitcast_convert_type(_bf(y[:, :HH]).astype(jnp.float32),
                                   jnp.int32)
    ihi = lax.bitcast_convert_type(_bf(y[:, HH:]).astype(jnp.float32),
                                   jnp.int32)
    packed = (ihi & _MASK_HI) | lax.shift_right_logical(ilo, 16)
    tab_ref[...] = lax.bitcast_convert_type(packed, jnp.float32)

    h = jnp.dot(xb, w1_ref[...], preferred_element_type=jnp.float32) + b1_ref[...]
    h = _bf(h)
    h = h * jax.nn.sigmoid(h)
    z = jnp.dot(h, w2_ref[...], preferred_element_type=jnp.float32) + b2_ref[...]
    go_ref[...] = x + _ln(z, g_ref[...], b_ref[...])


def _pregrid(x, wm, w1, b1, w2, b2, g, b):
    blk = 2000
    vspec = lambda r, c: pl.BlockSpec((r, c), lambda i: (0, 0))
    return pl.pallas_call(
        _pregrid_body,
        grid=(NG // blk,),
        in_specs=[
            pl.BlockSpec((blk, D), lambda i: (i, 0)),
            vspec(D, H), vspec(D, H), vspec(1, H), vspec(H, D), vspec(1, D),
            vspec(1, D), vspec(1, D),
        ],
        out_specs=(pl.BlockSpec((blk, HH), lambda i: (i, 0)),
                   pl.BlockSpec((blk, D), lambda i: (i, 0))),
        out_shape=(jax.ShapeDtypeStruct((NG, HH), jnp.float32),
                   jax.ShapeDtypeStruct((NG, D), jnp.float32)),
    )(x, wm, w1, b1, w2, b2, g, b)


def _edge_core(x_ref, agp_ref, dst_ref, xp_ref, w1c_ref, b1e_ref, w1_ref,
               w2_ref, b2_ref, g_ref, b_ref, agg_ref, bg_ref, tabb_ref, nb):
    i = pl.program_id(0)

    @pl.when(i == 0)
    def _():
        agg_ref[...] = jnp.zeros_like(agg_ref)
        tabb_ref[...] = _bf(jnp.dot(_bf(xp_ref[...]), w1c_ref[...],
                                    preferred_element_type=jnp.float32)
                            + b1e_ref[...])

    dstv = dst_ref[0]
    d_lo = jnp.min(dstv)
    d_hi = jnp.max(dstv)
    start = (d_lo // 16) * 16
    nchunk = (d_hi - start) // SEG_W + 1

    def exp_chunk(c, _):
        row0 = start + c * SEG_W
        rows = lax.broadcasted_iota(jnp.int32, (SEG_W, BE), 0) + row0
        oh = _bf(rows == dstv)
        band = tabb_ref[pl.ds(row0, SEG_W), :]
        contrib = _bf(lax.dot_general(oh, band, (((0,), (0,)), ((), ())),
                                      preferred_element_type=jnp.float32))

        @pl.when(c == 0)
        def _():
            bg_ref[...] = contrib

        @pl.when(c > 0)
        def _():
            bg_ref[...] += contrib

        return 0

    lax.fori_loop(0, nchunk, exp_chunk, 0)

    w = lax.bitcast_convert_type(agp_ref[...], jnp.int32)
    ag = _bf(jnp.concatenate(
        [lax.bitcast_convert_type(lax.shift_left(w, 16), jnp.float32),
         lax.bitcast_convert_type(w & _MASK_HI, jnp.float32)], axis=1))

    xb = _bf(x_ref[...])
    h = (_bf(jnp.dot(xb, w1_ref[...], preferred_element_type=jnp.float32))
         + ag + bg_ref[...])
    h = h * jax.nn.sigmoid(h)
    y = jnp.dot(h, w2_ref[...], preferred_element_type=jnp.float32) + b2_ref[...]
    ef = _bf(_ln(y, g_ref[...], b_ref[...]))

    def agg_chunk(c, _):
        row0 = start + c * SEG_W
        rows = lax.broadcasted_iota(jnp.int32, (SEG_W, BE), 0) + row0
        oh = _bf(rows == dstv)
        part = jnp.dot(oh, ef, preferred_element_type=jnp.float32)
        agg_ref[pl.ds(row0, SEG_W), :] += part
        return 0

    lax.fori_loop(0, nchunk, agg_chunk, 0)


def _edge0_body(x_ref, agp_ref, dst_ref, xp_ref, w1c_ref, b1e_ref, w1_ref,
                w2_ref, b2_ref, g_ref, b_ref, agg_ref, bg_ref, tabb_ref):
    _edge_core(x_ref, agp_ref, dst_ref, xp_ref, w1c_ref, b1e_ref, w1_ref,
               w2_ref, b2_ref, g_ref, b_ref, agg_ref, bg_ref, tabb_ref,
               EH0 // BE)


def _edge1_body(x_ref, agp_ref, dst_ref, xp_ref, w1c_ref, b1e_ref, w1_ref,
                w2_ref, b2_ref, g_ref, b_ref, a0_ref, w1da_ref, w1db_ref,
                b1d_ref, w2d_ref, b2d_ref, gd_ref, bd_ref, mesh_ref,
                bg_ref, tabb_ref, agg_ref):
    nb = EH1 // BE
    _edge_core(x_ref, agp_ref, dst_ref, xp_ref, w1c_ref, b1e_ref, w1_ref,
               w2_ref, b2_ref, g_ref, b_ref, agg_ref, bg_ref, tabb_ref, nb)

    @pl.when(pl.program_id(0) == nb - 1)
    def _():
        x = xp_ref[...]
        agg = agg_ref[...] + a0_ref[...]
        h = (jnp.dot(_bf(x), w1da_ref[...], preferred_element_type=jnp.float32)
             + jnp.dot(_bf(agg), w1db_ref[...],
                       preferred_element_type=jnp.float32)
             + b1d_ref[...])
        h = _bf(h)
        h = h * jax.nn.sigmoid(h)
        y = (jnp.dot(h, w2d_ref[...], preferred_element_type=jnp.float32)
             + b2d_ref[...])
        mesh_ref[...] = x + _ln(y, gd_ref[...], bd_ref[...])


def _edge0(g2m, agp, dst3, xpad, w1c, b1e, w1, w2, b2, g, b):
    vspec = lambda r, c: pl.BlockSpec((r, c), lambda i: (0, 0))
    return pl.pallas_call(
        _edge0_body,
        grid=(EH0 // BE,),
        in_specs=[
            pl.BlockSpec((BE, D), lambda i: (i, 0)),
            pl.BlockSpec((BE, HH), lambda i: (i, 0)),
            pl.BlockSpec((1, 1, BE), lambda i: (i, 0, 0)),
            vspec(MPAD, D), vspec(D, H), vspec(1, H),
            vspec(D, H), vspec(H, D), vspec(1, D), vspec(1, D), vspec(1, D),
        ],
        out_specs=pl.BlockSpec((MPAD, D), lambda i: (0, 0)),
        out_shape=jax.ShapeDtypeStruct((MPAD, D), jnp.float32),
        scratch_shapes=[pltpu.VMEM((BE, D), jnp.bfloat16),
                        pltpu.VMEM((MPAD, D), jnp.bfloat16)],
    )(g2m, agp, dst3, xpad, w1c, b1e, w1, w2, b2, g, b)


def _edge1(g2m, agp, dst3, xpad, w1c, b1e, w1, w2, b2, g, b,
           a0, w1da, w1db, b1d, w2d, b2d, gd, bd):
    vspec = lambda r, c: pl.BlockSpec((r, c), lambda i: (0, 0))
    off = EH0 // BE
    return pl.pallas_call(
        _edge1_body,
        grid=(EH1 // BE,),
        in_specs=[
            pl.BlockSpec((BE, D), lambda i: (i + off, 0)),
            pl.BlockSpec((BE, HH), lambda i: (i, 0)),
            pl.BlockSpec((1, 1, BE), lambda i: (i + off, 0, 0)),
            vspec(MPAD, D), vspec(D, H), vspec(1, H),
            vspec(D, H), vspec(H, D), vspec(1, D), vspec(1, D), vspec(1, D),
            vspec(MPAD, D),
            vspec(D, H), vspec(D, H), vspec(1, H), vspec(H, D), vspec(1, D),
            vspec(1, D), vspec(1, D),
        ],
        out_specs=pl.BlockSpec((MPAD, D), lambda i: (0, 0)),
        out_shape=jax.ShapeDtypeStruct((MPAD, D), jnp.float32),
        scratch_shapes=[pltpu.VMEM((BE, D), jnp.bfloat16),
                        pltpu.VMEM((MPAD, D), jnp.bfloat16),
                        pltpu.VMEM((MPAD, D), jnp.float32)],
    )(g2m, agp, dst3, xpad, w1c, b1e, w1, w2, b2, g, b,
      a0, w1da, w1db, b1d, w2d, b2d, gd, bd)


def kernel(g2m_efeat, grid_nfeat, mesh_nfeat, src_idx, dst_idx,
           W1e, b1e, W2e, b2e, ge, be,
           W1d, b1d, W2d, b2d, gd, bd,
           W1s, b1s, W2s, b2s, gs, bs):
    r1 = lambda v: v.reshape(1, -1)
    bf = lambda w: w.astype(jnp.bfloat16)

    xpad = jnp.pad(mesh_nfeat, ((0, MPAD - NM), (0, 0)))

    tab_a, grid_out = _pregrid(grid_nfeat, bf(W1e[D:2 * D]), bf(W1s),
                               r1(b1s), bf(W2s), r1(b2s), r1(gs), r1(bs))

    agp0 = _sc_gather(tab_a, src_idx, 0, EH0)
    agp1 = _sc_gather(tab_a, src_idx, EH0, EH1)

    dst3 = dst_idx.reshape(NE // BE, 1, BE)
    w1c, b1er = bf(W1e[2 * D:]), r1(b1e)
    w1a, w2 = bf(W1e[:D]), bf(W2e)
    b2r, ger, ber = r1(b2e), r1(ge), r1(be)
    agg0 = _edge0(g2m_efeat, agp0, dst3, xpad, w1c, b1er, w1a, w2, b2r,
                  ger, ber)
    mesh_out = _edge1(g2m_efeat, agp1, dst3, xpad, w1c, b1er, w1a, w2, b2r,
                      ger, ber, agg0, bf(W1d[:D]), bf(W1d[D:]), r1(b1d),
                      bf(W2d), r1(b2d), r1(gd), r1(bd))[:NM]
    return (grid_out, mesh_out)


# speedup vs baseline: 7.0051x; 1.1219x over previous
import functools

import jax
import jax.numpy as jnp
import numpy as np
from jax import lax
from jax.experimental import pallas as pl
from jax.experimental.pallas import tpu as pltpu
from jax.experimental.pallas import tpu_sc as plsc

NG = 10000
NM = 2500
NE = 160000
D = 256
H = 256
HH = H // 2

SEG_W = 96
MPAD = 2496 + SEG_W

NC = 2
NS = 16
NW = NC * NS

GBLK = 200

BE = 4000
EH0 = 64000
EH1 = NE - EH0

_MASK_HI = -65536


def _sc_gather(tab, src_idx, off, n_edges):
    per_w = n_edges // NW
    nblk = per_w // GBLK

    @functools.partial(
        pl.kernel,
        mesh=plsc.VectorSubcoreMesh(core_axis_name="c", subcore_axis_name="s"),
        out_type=jax.ShapeDtypeStruct((n_edges, HH), jnp.float32),
        scratch_types=[
            pltpu.VMEM((GBLK,), jnp.int32),
            pltpu.VMEM((GBLK,), jnp.int32),
            pltpu.VMEM((GBLK, HH), jnp.float32),
            pltpu.VMEM((GBLK, HH), jnp.float32),
            pltpu.SemaphoreType.DMA,
            pltpu.SemaphoreType.DMA,
        ],
    )
    def k(a_hbm, si_hbm, oa_hbm, idx0, idx1, r0, r1, sg0, sg1):
        wid = lax.axis_index("s") * NC + lax.axis_index("c")
        base = wid * per_w
        pltpu.sync_copy(si_hbm.at[pl.ds(off + base, GBLK)], idx0)
        pltpu.async_copy(a_hbm.at[idx0], r0, sg0)
        pltpu.sync_copy(si_hbm.at[pl.ds(off + base + GBLK, GBLK)], idx1)
        pltpu.async_copy(a_hbm.at[idx1], r1, sg1)

        @pl.loop(0, nblk // 2)
        def _(p):
            o = base + 2 * p * GBLK
            pltpu.make_async_copy(a_hbm.at[idx0], r0, sg0).wait()
            pltpu.sync_copy(r0, oa_hbm.at[pl.ds(o, GBLK)])

            @pl.when(2 * p + 2 < nblk)
            def _():
                pltpu.sync_copy(si_hbm.at[pl.ds(off + o + 2 * GBLK, GBLK)],
                                idx0)
                pltpu.async_copy(a_hbm.at[idx0], r0, sg0)

            pltpu.make_async_copy(a_hbm.at[idx1], r1, sg1).wait()
            pltpu.sync_copy(r1, oa_hbm.at[pl.ds(o + GBLK, GBLK)])

            @pl.when(2 * p + 3 < nblk)
            def _():
                pltpu.sync_copy(si_hbm.at[pl.ds(off + o + 3 * GBLK, GBLK)],
                                idx1)
                pltpu.async_copy(a_hbm.at[idx1], r1, sg1)

        if nblk % 2:
            o = base + (nblk - 1) * GBLK
            pltpu.make_async_copy(a_hbm.at[idx0], r0, sg0).wait()
            pltpu.sync_copy(r0, oa_hbm.at[pl.ds(o, GBLK)])

    return k(tab, src_idx)


def _bf(x):
    return x.astype(jnp.bfloat16)


def _ln(y, g, b):
    mu = jnp.mean(y, axis=-1, keepdims=True)
    d = y - mu
    var = jnp.mean(d * d, axis=-1, keepdims=True)
    return d * lax.rsqrt(var + 1e-5) * g + b


def _pregrid_body(x_ref, wm_ref, w1_ref, b1_ref, w2_ref, b2_ref, g_ref,
                  b_ref, tab_ref, go_ref):
    x = x_ref[...]
    xb = _bf(x)
    y = jnp.dot(xb, wm_ref[...], preferred_element_type=jnp.float32)
    ilo = lax.bitcast_convert_type(_bf(y[:, :HH]).astype(jnp.float32),
                                   jnp.int32)
    ihi = lax.bitcast_convert_type(_bf(y[:, HH:]).astype(jnp.float32),
                                   jnp.int32)
    packed = (ihi & _MASK_HI) | lax.shift_right_logical(ilo, 16)
    tab_ref[...] = lax.bitcast_convert_type(packed, jnp.float32)

    h = jnp.dot(xb, w1_ref[...], preferred_element_type=jnp.float32) + b1_ref[...]
    h = _bf(h)
    h = h * jax.nn.sigmoid(h)
    z = jnp.dot(h, w2_ref[...], preferred_element_type=jnp.float32) + b2_ref[...]
    go_ref[...] = x + _ln(z, g_ref[...], b_ref[...])


def _pregrid(x, wm, w1, b1, w2, b2, g, b):
    blk = 2000
    vspec = lambda r, c: pl.BlockSpec((r, c), lambda i: (0, 0))
    return pl.pallas_call(
        _pregrid_body,
        grid=(NG // blk,),
        in_specs=[
            pl.BlockSpec((blk, D), lambda i: (i, 0)),
            vspec(D, H), vspec(D, H), vspec(1, H), vspec(H, D), vspec(1, D),
            vspec(1, D), vspec(1, D),
        ],
        out_specs=(pl.BlockSpec((blk, HH), lambda i: (i, 0)),
                   pl.BlockSpec((blk, D), lambda i: (i, 0))),
        out_shape=(jax.ShapeDtypeStruct((NG, HH), jnp.float32),
                   jax.ShapeDtypeStruct((NG, D), jnp.float32)),
    )(x, wm, w1, b1, w2, b2, g, b)


def _edge_core(x_ref, agp_ref, dst_ref, xp_ref, w1c_ref, b1e_ref, w1_ref,
               w2_ref, b2_ref, g_ref, b_ref, agg_ref, bg_ref, tabb_ref, nb):
    i = pl.program_id(0)

    @pl.when(i == 0)
    def _():
        agg_ref[...] = jnp.zeros_like(agg_ref)
        tabb_ref[...] = _bf(jnp.dot(_bf(xp_ref[...]), w1c_ref[...],
                                    preferred_element_type=jnp.float32)
                            + b1e_ref[...])

    dstv = dst_ref[0]
    d_lo = jnp.min(dstv)
    d_hi = jnp.max(dstv)
    start = (d_lo // 16) * 16
    nchunk = (d_hi - start) // SEG_W + 1

    def exp_chunk(c, _):
        row0 = start + c * SEG_W
        rows = lax.broadcasted_iota(jnp.int32, (SEG_W, BE), 0) + row0
        oh = _bf(rows == dstv)
        band = tabb_ref[pl.ds(row0, SEG_W), :]
        contrib = _bf(lax.dot_general(oh, band, (((0,), (0,)), ((), ())),
                                      preferred_element_type=jnp.float32))

        @pl.when(c == 0)
        def _():
            bg_ref[...] = contrib

        @pl.when(c > 0)
        def _():
            bg_ref[...] += contrib

        return 0

    lax.fori_loop(0, nchunk, exp_chunk, 0)

    w = lax.bitcast_convert_type(agp_ref[...], jnp.int32)
    ag = _bf(jnp.concatenate(
        [lax.bitcast_convert_type(lax.shift_left(w, 16), jnp.float32),
         lax.bitcast_convert_type(w & _MASK_HI, jnp.float32)], axis=1))

    xb = _bf(x_ref[...])
    h = (_bf(jnp.dot(xb, w1_ref[...], preferred_element_type=jnp.float32))
         + ag + bg_ref[...])
    h = h * jax.nn.sigmoid(h)
    y = jnp.dot(h, w2_ref[...], preferred_element_type=jnp.float32) + b2_ref[...]
    ef = _bf(_ln(y, g_ref[...], b_ref[...]))

    def agg_chunk(c, _):
        row0 = start + c * SEG_W
        rows = lax.broadcasted_iota(jnp.int32, (SEG_W, BE), 0) + row0
        oh = _bf(rows == dstv)
        part = jnp.dot(oh, ef, preferred_element_type=jnp.float32)
        agg_ref[pl.ds(row0, SEG_W), :] += part
        return 0

    lax.fori_loop(0, nchunk, agg_chunk, 0)


def _edge0_body(x_ref, agp_ref, dst_ref, xp_ref, w1c_ref, b1e_ref, w1_ref,
                w2_ref, b2_ref, g_ref, b_ref, agg_ref, bg_ref, tabb_ref):
    _edge_core(x_ref, agp_ref, dst_ref, xp_ref, w1c_ref, b1e_ref, w1_ref,
               w2_ref, b2_ref, g_ref, b_ref, agg_ref, bg_ref, tabb_ref,
               EH0 // BE)


def _edge1_body(x_ref, agp_ref, dst_ref, xp_ref, w1c_ref, b1e_ref, w1_ref,
                w2_ref, b2_ref, g_ref, b_ref, a0_ref, w1da_ref, w1db_ref,
                b1d_ref, w2d_ref, b2d_ref, gd_ref, bd_ref, mesh_ref,
                bg_ref, tabb_ref, agg_ref):
    nb = EH1 // BE
    _edge_core(x_ref, agp_ref, dst_ref, xp_ref, w1c_ref, b1e_ref, w1_ref,
               w2_ref, b2_ref, g_ref, b_ref, agg_ref, bg_ref, tabb_ref, nb)

    @pl.when(pl.program_id(0) == nb - 1)
    def _():
        x = xp_ref[...]
        agg = agg_ref[...] + a0_ref[...]
        h = (jnp.dot(_bf(x), w1da_ref[...], preferred_element_type=jnp.float32)
             + jnp.dot(_bf(agg), w1db_ref[...],
                       preferred_element_type=jnp.float32)
             + b1d_ref[...])
        h = _bf(h)
        h = h * jax.nn.sigmoid(h)
        y = (jnp.dot(h, w2d_ref[...], preferred_element_type=jnp.float32)
             + b2d_ref[...])
        mesh_ref[...] = x + _ln(y, gd_ref[...], bd_ref[...])


def _edge0(g2m, agp, dst3, xpad, w1c, b1e, w1, w2, b2, g, b):
    vspec = lambda r, c: pl.BlockSpec((r, c), lambda i: (0, 0))
    return pl.pallas_call(
        _edge0_body,
        grid=(EH0 // BE,),
        in_specs=[
            pl.BlockSpec((BE, D), lambda i: (i, 0)),
            pl.BlockSpec((BE, HH), lambda i: (i, 0)),
            pl.BlockSpec((1, 1, BE), lambda i: (i, 0, 0)),
            vspec(MPAD, D), vspec(D, H), vspec(1, H),
            vspec(D, H), vspec(H, D), vspec(1, D), vspec(1, D), vspec(1, D),
        ],
        out_specs=pl.BlockSpec((MPAD, D), lambda i: (0, 0)),
        out_shape=jax.ShapeDtypeStruct((MPAD, D), jnp.float32),
        scratch_shapes=[pltpu.VMEM((BE, D), jnp.bfloat16),
                        pltpu.VMEM((MPAD, D), jnp.bfloat16)],
    )(g2m, agp, dst3, xpad, w1c, b1e, w1, w2, b2, g, b)


def _edge1(g2m, agp, dst3, xpad, w1c, b1e, w1, w2, b2, g, b,
           a0, w1da, w1db, b1d, w2d, b2d, gd, bd):
    vspec = lambda r, c: pl.BlockSpec((r, c), lambda i: (0, 0))
    off = EH0 // BE
    return pl.pallas_call(
        _edge1_body,
        grid=(EH1 // BE,),
        in_specs=[
            pl.BlockSpec((BE, D), lambda i: (i + off, 0)),
            pl.BlockSpec((BE, HH), lambda i: (i, 0)),
            pl.BlockSpec((1, 1, BE), lambda i: (i + off, 0, 0)),
            vspec(MPAD, D), vspec(D, H), vspec(1, H),
            vspec(D, H), vspec(H, D), vspec(1, D), vspec(1, D), vspec(1, D),
            vspec(MPAD, D),
            vspec(D, H), vspec(D, H), vspec(1, H), vspec(H, D), vspec(1, D),
            vspec(1, D), vspec(1, D),
        ],
        out_specs=pl.BlockSpec((MPAD, D), lambda i: (0, 0)),
        out_shape=jax.ShapeDtypeStruct((MPAD, D), jnp.float32),
        scratch_shapes=[pltpu.VMEM((BE, D), jnp.bfloat16),
                        pltpu.VMEM((MPAD, D), jnp.bfloat16),
                        pltpu.VMEM((MPAD, D), jnp.float32)],
    )(g2m, agp, dst3, xpad, w1c, b1e, w1, w2, b2, g, b,
      a0, w1da, w1db, b1d, w2d, b2d, gd, bd)


def kernel(g2m_efeat, grid_nfeat, mesh_nfeat, src_idx, dst_idx,
           W1e, b1e, W2e, b2e, ge, be,
           W1d, b1d, W2d, b2d, gd, bd,
           W1s, b1s, W2s, b2s, gs, bs):
    r1 = lambda v: v.reshape(1, -1)
    bf = lambda w: w.astype(jnp.bfloat16)

    xpad = jnp.pad(mesh_nfeat, ((0, MPAD - NM), (0, 0)))

    tab_a, grid_out = _pregrid(grid_nfeat, bf(W1e[D:2 * D]), bf(W1s),
                               r1(b1s), bf(W2s), r1(b2s), r1(gs), r1(bs))

    agp0 = _sc_gather(tab_a, src_idx, 0, EH0)
    agp1 = _sc_gather(tab_a, src_idx, EH0, EH1)

    dst3 = dst_idx.reshape(NE // BE, 1, BE)
    w1c, b1er = bf(W1e[2 * D:]), r1(b1e)
    w1a, w2 = bf(W1e[:D]), bf(W2e)
    b2r, ger, ber = r1(b2e), r1(ge), r1(be)
    agg0 = _edge0(g2m_efeat, agp0, dst3, xpad, w1c, b1er, w1a, w2, b2r,
                  ger, ber)
    mesh_out = _edge1(g2m_efeat, agp1, dst3, xpad, w1c, b1er, w1a, w2, b2r,
                      ger, ber, agg0, bf(W1d[:D]), bf(W1d[D:]), r1(b1d),
                      bf(W2d), r1(b2d), r1(gd), r1(bd))[:NM]
    return (grid_out, mesh_out)


# speedup vs baseline: 7.3145x; 1.0442x over previous
import functools

import jax
import jax.numpy as jnp
import numpy as np
from jax import lax
from jax.experimental import pallas as pl
from jax.experimental.pallas import tpu as pltpu
from jax.experimental.pallas import tpu_sc as plsc

NG = 10000
NM = 2500
NE = 160000
D = 256
H = 256
HH = H // 2

SEG_W = 144
MPAD = 2496 + SEG_W

NC = 2
NS = 16
NW = NC * NS

GBLK = 200

BE = 8000
EH0 = 64000
EH1 = NE - EH0

_MASK_HI = -65536


def _sc_gather(tab, src_idx, off, n_edges):
    per_w = n_edges // NW
    nblk = per_w // GBLK

    @functools.partial(
        pl.kernel,
        mesh=plsc.VectorSubcoreMesh(core_axis_name="c", subcore_axis_name="s"),
        out_type=jax.ShapeDtypeStruct((n_edges, HH), jnp.float32),
        scratch_types=[
            pltpu.VMEM((GBLK,), jnp.int32),
            pltpu.VMEM((GBLK,), jnp.int32),
            pltpu.VMEM((GBLK, HH), jnp.float32),
            pltpu.VMEM((GBLK, HH), jnp.float32),
            pltpu.SemaphoreType.DMA,
            pltpu.SemaphoreType.DMA,
        ],
    )
    def k(a_hbm, si_hbm, oa_hbm, idx0, idx1, r0, r1, sg0, sg1):
        wid = lax.axis_index("s") * NC + lax.axis_index("c")
        base = wid * per_w
        pltpu.sync_copy(si_hbm.at[pl.ds(off + base, GBLK)], idx0)
        pltpu.async_copy(a_hbm.at[idx0], r0, sg0)
        pltpu.sync_copy(si_hbm.at[pl.ds(off + base + GBLK, GBLK)], idx1)
        pltpu.async_copy(a_hbm.at[idx1], r1, sg1)

        @pl.loop(0, nblk // 2)
        def _(p):
            o = base + 2 * p * GBLK
            pltpu.make_async_copy(a_hbm.at[idx0], r0, sg0).wait()
            pltpu.sync_copy(r0, oa_hbm.at[pl.ds(o, GBLK)])

            @pl.when(2 * p + 2 < nblk)
            def _():
                pltpu.sync_copy(si_hbm.at[pl.ds(off + o + 2 * GBLK, GBLK)],
                                idx0)
                pltpu.async_copy(a_hbm.at[idx0], r0, sg0)

            pltpu.make_async_copy(a_hbm.at[idx1], r1, sg1).wait()
            pltpu.sync_copy(r1, oa_hbm.at[pl.ds(o + GBLK, GBLK)])

            @pl.when(2 * p + 3 < nblk)
            def _():
                pltpu.sync_copy(si_hbm.at[pl.ds(off + o + 3 * GBLK, GBLK)],
                                idx1)
                pltpu.async_copy(a_hbm.at[idx1], r1, sg1)

        if nblk % 2:
            o = base + (nblk - 1) * GBLK
            pltpu.make_async_copy(a_hbm.at[idx0], r0, sg0).wait()
            pltpu.sync_copy(r0, oa_hbm.at[pl.ds(o, GBLK)])

    return k(tab, src_idx)


def _bf(x):
    return x.astype(jnp.bfloat16)


def _ln(y, g, b):
    mu = jnp.mean(y, axis=-1, keepdims=True)
    d = y - mu
    var = jnp.mean(d * d, axis=-1, keepdims=True)
    return d * lax.rsqrt(var + 1e-5) * g + b


def _pregrid_body(x_ref, wm_ref, w1_ref, b1_ref, w2_ref, b2_ref, g_ref,
                  b_ref, tab_ref, go_ref):
    x = x_ref[...]
    xb = _bf(x)
    y = jnp.dot(xb, wm_ref[...], preferred_element_type=jnp.float32)
    ilo = lax.bitcast_convert_type(_bf(y[:, :HH]).astype(jnp.float32),
                                   jnp.int32)
    ihi = lax.bitcast_convert_type(_bf(y[:, HH:]).astype(jnp.float32),
                                   jnp.int32)
    packed = (ihi & _MASK_HI) | lax.shift_right_logical(ilo, 16)
    tab_ref[...] = lax.bitcast_convert_type(packed, jnp.float32)

    h = jnp.dot(xb, w1_ref[...], preferred_element_type=jnp.float32) + b1_ref[...]
    h = _bf(h)
    h = h * jax.nn.sigmoid(h)
    z = jnp.dot(h, w2_ref[...], preferred_element_type=jnp.float32) + b2_ref[...]
    go_ref[...] = x + _ln(z, g_ref[...], b_ref[...])


def _pregrid(x, wm, w1, b1, w2, b2, g, b):
    blk = 2000
    vspec = lambda r, c: pl.BlockSpec((r, c), lambda i: (0, 0))
    return pl.pallas_call(
        _pregrid_body,
        grid=(NG // blk,),
        in_specs=[
            pl.BlockSpec((blk, D), lambda i: (i, 0)),
            vspec(D, H), vspec(D, H), vspec(1, H), vspec(H, D), vspec(1, D),
            vspec(1, D), vspec(1, D),
        ],
        out_specs=(pl.BlockSpec((blk, HH), lambda i: (i, 0)),
                   pl.BlockSpec((blk, D), lambda i: (i, 0))),
        out_shape=(jax.ShapeDtypeStruct((NG, HH), jnp.float32),
                   jax.ShapeDtypeStruct((NG, D), jnp.float32)),
    )(x, wm, w1, b1, w2, b2, g, b)


def _edge_core(x_ref, agp_ref, dst_ref, xp_ref, w1c_ref, b1e_ref, w1_ref,
               w2_ref, b2_ref, g_ref, b_ref, agg_ref, bg_ref, tabb_ref, nb):
    i = pl.program_id(0)

    @pl.when(i == 0)
    def _():
        agg_ref[...] = jnp.zeros_like(agg_ref)
        tabb_ref[...] = _bf(jnp.dot(_bf(xp_ref[...]), w1c_ref[...],
                                    preferred_element_type=jnp.float32)
                            + b1e_ref[...])

    dstv = dst_ref[0]
    d_lo = jnp.min(dstv)
    d_hi = jnp.max(dstv)
    start = (d_lo // 16) * 16
    nchunk = (d_hi - start) // SEG_W + 1

    def exp_chunk(c, _):
        row0 = start + c * SEG_W
        rows = lax.broadcasted_iota(jnp.int32, (SEG_W, BE), 0) + row0
        oh = _bf(rows == dstv)
        band = tabb_ref[pl.ds(row0, SEG_W), :]
        contrib = _bf(lax.dot_general(oh, band, (((0,), (0,)), ((), ())),
                                      preferred_element_type=jnp.float32))

        @pl.when(c == 0)
        def _():
            bg_ref[...] = contrib

        @pl.when(c > 0)
        def _():
            bg_ref[...] += contrib

        return 0

    lax.fori_loop(0, nchunk, exp_chunk, 0)

    w = lax.bitcast_convert_type(agp_ref[...], jnp.int32)
    ag = _bf(jnp.concatenate(
        [lax.bitcast_convert_type(lax.shift_left(w, 16), jnp.float32),
         lax.bitcast_convert_type(w & _MASK_HI, jnp.float32)], axis=1))

    xb = _bf(x_ref[...])
    h = (_bf(jnp.dot(xb, w1_ref[...], preferred_element_type=jnp.float32))
         + ag + bg_ref[...])
    h = h * jax.nn.sigmoid(h)
    y = jnp.dot(h, w2_ref[...], preferred_element_type=jnp.float32) + b2_ref[...]
    ef = _bf(_ln(y, g_ref[...], b_ref[...]))

    def agg_chunk(c, _):
        row0 = start + c * SEG_W
        rows = lax.broadcasted_iota(jnp.int32, (SEG_W, BE), 0) + row0
        oh = _bf(rows == dstv)
        part = jnp.dot(oh, ef, preferred_element_type=jnp.float32)
        agg_ref[pl.ds(row0, SEG_W), :] += part
        return 0

    lax.fori_loop(0, nchunk, agg_chunk, 0)


def _edge0_body(x_ref, agp_ref, dst_ref, xp_ref, w1c_ref, b1e_ref, w1_ref,
                w2_ref, b2_ref, g_ref, b_ref, agg_ref, bg_ref, tabb_ref):
    _edge_core(x_ref, agp_ref, dst_ref, xp_ref, w1c_ref, b1e_ref, w1_ref,
               w2_ref, b2_ref, g_ref, b_ref, agg_ref, bg_ref, tabb_ref,
               EH0 // BE)


def _edge1_body(x_ref, agp_ref, dst_ref, xp_ref, w1c_ref, b1e_ref, w1_ref,
                w2_ref, b2_ref, g_ref, b_ref, a0_ref, w1da_ref, w1db_ref,
                b1d_ref, w2d_ref, b2d_ref, gd_ref, bd_ref, mesh_ref,
                bg_ref, tabb_ref, agg_ref):
    nb = EH1 // BE
    _edge_core(x_ref, agp_ref, dst_ref, xp_ref, w1c_ref, b1e_ref, w1_ref,
               w2_ref, b2_ref, g_ref, b_ref, agg_ref, bg_ref, tabb_ref, nb)

    @pl.when(pl.program_id(0) == nb - 1)
    def _():
        x = xp_ref[...]
        agg = agg_ref[...] + a0_ref[...]
        h = (jnp.dot(_bf(x), w1da_ref[...], preferred_element_type=jnp.float32)
             + jnp.dot(_bf(agg), w1db_ref[...],
                       preferred_element_type=jnp.float32)
             + b1d_ref[...])
        h = _bf(h)
        h = h * jax.nn.sigmoid(h)
        y = (jnp.dot(h, w2d_ref[...], preferred_element_type=jnp.float32)
             + b2d_ref[...])
        mesh_ref[...] = x + _ln(y, gd_ref[...], bd_ref[...])


def _edge0(g2m, agp, dst3, xpad, w1c, b1e, w1, w2, b2, g, b):
    vspec = lambda r, c: pl.BlockSpec((r, c), lambda i: (0, 0))
    return pl.pallas_call(
        _edge0_body,
        grid=(EH0 // BE,),
        in_specs=[
            pl.BlockSpec((BE, D), lambda i: (i, 0)),
            pl.BlockSpec((BE, HH), lambda i: (i, 0)),
            pl.BlockSpec((1, 1, BE), lambda i: (i, 0, 0)),
            vspec(MPAD, D), vspec(D, H), vspec(1, H),
            vspec(D, H), vspec(H, D), vspec(1, D), vspec(1, D), vspec(1, D),
        ],
        out_specs=pl.BlockSpec((MPAD, D), lambda i: (0, 0)),
        out_shape=jax.ShapeDtypeStruct((MPAD, D), jnp.float32),
        scratch_shapes=[pltpu.VMEM((BE, D), jnp.bfloat16),
                        pltpu.VMEM((MPAD, D), jnp.bfloat16)],
    )(g2m, agp, dst3, xpad, w1c, b1e, w1, w2, b2, g, b)


def _edge1(g2m, agp, dst3, xpad, w1c, b1e, w1, w2, b2, g, b,
           a0, w1da, w1db, b1d, w2d, b2d, gd, bd):
    vspec = lambda r, c: pl.BlockSpec((r, c), lambda i: (0, 0))
    off = EH0 // BE
    return pl.pallas_call(
        _edge1_body,
        grid=(EH1 // BE,),
        in_specs=[
            pl.BlockSpec((BE, D), lambda i: (i + off, 0)),
            pl.BlockSpec((BE, HH), lambda i: (i, 0)),
            pl.BlockSpec((1, 1, BE), lambda i: (i + off, 0, 0)),
            vspec(MPAD, D), vspec(D, H), vspec(1, H),
            vspec(D, H), vspec(H, D), vspec(1, D), vspec(1, D), vspec(1, D),
            vspec(MPAD, D),
            vspec(D, H), vspec(D, H), vspec(1, H), vspec(H, D), vspec(1, D),
            vspec(1, D), vspec(1, D),
        ],
        out_specs=pl.BlockSpec((MPAD, D), lambda i: (0, 0)),
        out_shape=jax.ShapeDtypeStruct((MPAD, D), jnp.float32),
        scratch_shapes=[pltpu.VMEM((BE, D), jnp.bfloat16),
                        pltpu.VMEM((MPAD, D), jnp.bfloat16),
                        pltpu.VMEM((MPAD, D), jnp.float32)],
    )(g2m, agp, dst3, xpad, w1c, b1e, w1, w2, b2, g, b,
      a0, w1da, w1db, b1d, w2d, b2d, gd, bd)


def kernel(g2m_efeat, grid_nfeat, mesh_nfeat, src_idx, dst_idx,
           W1e, b1e, W2e, b2e, ge, be,
           W1d, b1d, W2d, b2d, gd, bd,
           W1s, b1s, W2s, b2s, gs, bs):
    r1 = lambda v: v.reshape(1, -1)
    bf = lambda w: w.astype(jnp.bfloat16)

    xpad = jnp.pad(mesh_nfeat, ((0, MPAD - NM), (0, 0)))

    tab_a, grid_out = _pregrid(grid_nfeat, bf(W1e[D:2 * D]), bf(W1s),
                               r1(b1s), bf(W2s), r1(b2s), r1(gs), r1(bs))

    agp0 = _sc_gather(tab_a, src_idx, 0, EH0)
    agp1 = _sc_gather(tab_a, src_idx, EH0, EH1)

    dst3 = dst_idx.reshape(NE // BE, 1, BE)
    w1c, b1er = bf(W1e[2 * D:]), r1(b1e)
    w1a, w2 = bf(W1e[:D]), bf(W2e)
    b2r, ger, ber = r1(b2e), r1(ge), r1(be)
    agg0 = _edge0(g2m_efeat, agp0, dst3, xpad, w1c, b1er, w1a, w2, b2r,
                  ger, ber)
    mesh_out = _edge1(g2m_efeat, agp1, dst3, xpad, w1c, b1er, w1a, w2, b2r,
                      ger, ber, agg0, bf(W1d[:D]), bf(W1d[D:]), r1(b1d),
                      bf(W2d), r1(b2d), r1(gd), r1(bd))[:NM]
    return (grid_out, mesh_out)
